# baseline (speedup 1.0000x reference)
import functools

import jax
import jax.numpy as jnp
from jax import lax
from jax.experimental import pallas as pl
from jax.experimental.pallas import tpu as pltpu
from jax.experimental.pallas import tpu_sc as plsc

N = 10000
D = 128
E = 320000
G = 2
ALPHA = 0.5
BETA = 0.5

NSUB = 16
CHUNK = 128
N_PAD = 10240
ROWS_PER_SUB = N_PAD // NSUB
JUNK = N + 100
E_PAD = -(-E // (8 * NSUB * CHUNK)) * (8 * NSUB * CHUNK)
NCH = E_PAD // CHUNK
CPS = NCH // NSUB
QN = 4
QCH = CPS // QN
QHALF = QCH // 2
PCHUNKS = N_PAD // CHUNK


def _sc_mesh():
    return plsc.VectorSubcoreMesh(
        core_axis_name="c", subcore_axis_name="s", num_cores=G
    )


def _spmm(table, src_r, dst_r):
    @functools.partial(
        pl.kernel,
        out_type=jax.ShapeDtypeStruct((G, N_PAD, D), jnp.float32),
        mesh=_sc_mesh(),
        scratch_types=[
            pltpu.VMEM((QCH, CHUNK), jnp.int32),
            pltpu.VMEM((QCH, CHUNK), jnp.int32),
            pltpu.VMEM((CHUNK, D), jnp.float32),
            pltpu.VMEM((CHUNK, D), jnp.float32),
            pltpu.VMEM_SHARED((N_PAD, D), jnp.float32),
            pltpu.SemaphoreType.DMA,
            pltpu.SemaphoreType.DMA,
            pltpu.SemaphoreType.DMA,
            pltpu.SemaphoreType.DMA,
        ],
    )
    def k(table_ref, src_ref, dst_ref, out_ref, idx_s, idx_d, r0, r1,
          acc, g0, g1, s0, s1):
        c = lax.axis_index("c")
        s = lax.axis_index("s")

        def zrow(r, carry):
            for j in range(D // 16):
                r0[r, pl.ds(j * 16, 16)] = jnp.zeros((16,), jnp.float32)
            return carry

        lax.fori_loop(0, CHUNK, zrow, 0)
        for t in range(ROWS_PER_SUB // CHUNK):
            pltpu.sync_copy(
                r0, acc.at[pl.ds(s * ROWS_PER_SUB + t * CHUNK, CHUNK)]
            )
        plsc.subcore_barrier()

        def body(i, carry):
            pltpu.async_copy(table_ref.at[idx_s.at[i]], r0, g0)
            pltpu.make_async_copy(
                table_ref.at[pl.ds(0, CHUNK)], r0, g0
            ).wait()
            pltpu.sync_copy(r0, acc.at[idx_d.at[i]], add=True)
            return carry

        for q in range(QN):
            base = s * CPS + q * QCH
            pltpu.sync_copy(src_ref.at[c, pl.ds(base, QCH)], idx_s)
            pltpu.sync_copy(dst_ref.at[c, pl.ds(base, QCH)], idx_d)
            lax.fori_loop(0, QCH, body, 0)
        plsc.subcore_barrier()
        pltpu.sync_copy(
            acc.at[pl.ds(s * ROWS_PER_SUB, ROWS_PER_SUB)],
            out_ref.at[c, pl.ds(s * ROWS_PER_SUB, ROWS_PER_SUB)],
        )

    return k(table, src_r, dst_r)


def _perm_gather(xt, ht, perm_r):
    NFULL = N // CHUNK
    NTAIL = N - NFULL * CHUNK

    @functools.partial(
        pl.kernel,
        out_type=(
            jax.ShapeDtypeStruct((N_PAD, D), jnp.float32),
            jax.ShapeDtypeStruct((N_PAD, D), jnp.float32),
            jax.ShapeDtypeStruct((N_PAD, D), jnp.float32),
        ),
        mesh=_sc_mesh(),
        scratch_types=[
            pltpu.VMEM((CHUNK,), jnp.int32),
            pltpu.VMEM((CHUNK, D), jnp.float32),
            pltpu.SemaphoreType.DMA,
        ],
    )
    def k(x_ref, h_ref, perm_ref, xp_ref, hp_ref, xlin_ref, idx, rows, sem):
        c = lax.axis_index("c")
        s = lax.axis_index("s")
        w = s * G + c
        for t in range(-(-PCHUNKS // (2 * NSUB))):
            cid = w + 2 * NSUB * t

            @pl.when(cid < PCHUNKS)
            def _():
                pltpu.sync_copy(perm_ref.at[cid], idx)
                pltpu.async_copy(x_ref.at[idx], rows, sem).wait()
                pltpu.sync_copy(rows, xp_ref.at[pl.ds(cid * CHUNK, CHUNK)])
                pltpu.async_copy(h_ref.at[idx], rows, sem).wait()
                pltpu.sync_copy(rows, hp_ref.at[pl.ds(cid * CHUNK, CHUNK)])

            @pl.when(cid < NFULL)
            def _():
                pltpu.sync_copy(x_ref.at[pl.ds(cid * CHUNK, CHUNK)], rows)
                pltpu.sync_copy(rows, xlin_ref.at[pl.ds(cid * CHUNK, CHUNK)])

            @pl.when(cid == NFULL)
            def _():
                pltpu.sync_copy(
                    x_ref.at[pl.ds(NFULL * CHUNK, NTAIL)],
                    rows.at[pl.ds(0, NTAIL)],
                )
                pltpu.sync_copy(
                    rows.at[pl.ds(0, NTAIL)],
                    xlin_ref.at[pl.ds(NFULL * CHUNK, NTAIL)],
                )

    return k(xt, ht, perm_r)


def _fc(feature, W1):
    blk = 2000

    def body(f_ref, w_ref, o_ref):
        o_ref[...] = jnp.dot(
            f_ref[...], w_ref[...], preferred_element_type=jnp.float32
        )

    return pl.pallas_call(
        body,
        grid=(N // blk,),
        in_specs=[
            pl.BlockSpec((blk, D), lambda i: (i, 0)),
            pl.BlockSpec((D, D), lambda i: (0, 0)),
        ],
        out_specs=pl.BlockSpec((blk, D), lambda i: (i, 0)),
        out_shape=jax.ShapeDtypeStruct((N, D), jnp.float32),
    )(feature, W1)


def _losses(temp, x, h1, h2, zp, xp, H, hp):
    blk = 2000
    grid_n = N // blk

    def body(t_ref, x_ref, h1a, h1b, h2a, h2b, zpa, zpb, xp_ref, hh_ref,
             hp_ref, loss_ref, reg_ref):
        i = pl.program_id(0)

        @pl.when(i == 0)
        def _():
            loss_ref[0, 0] = 0.0
            reg_ref[0, 0] = 0.0

        xv = x_ref[...]
        xpv = xp_ref[...]
        hh = hh_ref[...]
        hpv = hp_ref[...]
        dHf = jnp.sum((hh - hpv) ** 2, axis=1)
        lsum = jnp.float32(0.0)
        berns = []
        for g, (h1r, h2r, zpr) in enumerate(
            ((h1a, h2a, zpa), (h1b, h2b, zpb))
        ):
            a0 = jnp.maximum(t_ref[g, 0], 0.0) * 0.25
            a1 = jnp.maximum(t_ref[g, 1], 0.0) * 0.5
            a2 = jnp.maximum(t_ref[g, 2], 0.0) * 0.25
            bern = a0 * xv + a1 * h1r[0] + a2 * h2r[0]
            d1p = jnp.sum((bern - zpr[0]) ** 2, axis=1)
            d1n = jnp.sum((bern - xpv) ** 2, axis=1)
            lsum += jnp.sum(jnp.maximum(d1p - d1n + ALPHA, 0.0))
            d2p = jnp.sum((hh - bern) ** 2, axis=1)
            lsum += jnp.sum(jnp.maximum(d2p - dHf + BETA, 0.0))
            berns.append(bern)
        pos = 0.5 * (berns[0] + berns[1])
        loss_ref[0, 0] += lsum * jnp.float32(1.0 / N)
        reg_ref[0, 0] += jnp.sum((hh - pos) ** 2)

    g3 = lambda i: (0, i, 0)
    g3b = lambda i: (1, i, 0)
    loss2, reg2 = pl.pallas_call(
        body,
        grid=(grid_n,),
        in_specs=[
            pl.BlockSpec(memory_space=pltpu.SMEM),
            pl.BlockSpec((blk, D), lambda i: (i, 0)),
            pl.BlockSpec((1, blk, D), g3),
            pl.BlockSpec((1, blk, D), g3b),
            pl.BlockSpec((1, blk, D), g3),
            pl.BlockSpec((1, blk, D), g3b),
            pl.BlockSpec((1, blk, D), g3),
            pl.BlockSpec((1, blk, D), g3b),
            pl.BlockSpec((blk, D), lambda i: (i, 0)),
            pl.BlockSpec((blk, D), lambda i: (i, 0)),
            pl.BlockSpec((blk, D), lambda i: (i, 0)),
        ],
        out_specs=[
            pl.BlockSpec((1, 1), lambda i: (0, 0), memory_space=pltpu.SMEM),
            pl.BlockSpec((1, 1), lambda i: (0, 0), memory_space=pltpu.SMEM),
        ],
        out_shape=[
            jax.ShapeDtypeStruct((1, 1), jnp.float32),
            jax.ShapeDtypeStruct((1, 1), jnp.float32),
        ],
        compiler_params=pltpu.CompilerParams(
            dimension_semantics=("arbitrary",)
        ),
    )(temp, x, h1, h1, h2, h2, zp, zp, xp, H, hp)
    return loss2[0, 0], reg2[0, 0]


def _prep_edges(ei, src_off):
    src = ei[:, 0, :].astype(jnp.int32)
    dst = ei[:, 1, :].astype(jnp.int32)
    pad = E_PAD - E
    src = jnp.concatenate([src, jnp.zeros((G, pad), jnp.int32)], axis=1)
    dst = jnp.concatenate([dst, jnp.full((G, pad), JUNK, jnp.int32)], axis=1)
    if src_off:
        src = src + (jnp.arange(G, dtype=jnp.int32) * N_PAD)[:, None]
    return src.reshape(G, NCH, CHUNK), dst.reshape(G, NCH, CHUNK)


def kernel(feature, adj, neighbor_adj, sparse, msk, samp_bias1, samp_bias2,
           W1, temp, H, perm_idx):
    src_r, dst_r = _prep_edges(adj, False)
    src2_r, _ = _prep_edges(adj, True)
    nsrc_r, ndst_r = _prep_edges(neighbor_adj, False)
    perm_r = jnp.concatenate(
        [perm_idx.astype(jnp.int32), jnp.zeros((N_PAD - N,), jnp.int32)]
    ).reshape(PCHUNKS, CHUNK)

    x = _fc(feature, W1)
    xp, hp, xlin = _perm_gather(x, H, perm_r)
    h1 = _spmm(xlin, src_r, dst_r)
    zp = _spmm(xlin, nsrc_r, ndst_r)
    h2 = _spmm(h1.reshape(G * N_PAD, D), src2_r, dst_r)
    loss, reg = _losses(temp, x, h1, h2, zp, xp, H, hp)
    return loss, reg


# speedup vs baseline: 1.4401x; 1.4401x over previous
import functools

import jax
import jax.numpy as jnp
from jax import lax
from jax.experimental import pallas as pl
from jax.experimental.pallas import tpu as pltpu
from jax.experimental.pallas import tpu_sc as plsc

N = 10000
D = 128
E = 320000
G = 2
ALPHA = 0.5
BETA = 0.5

NSUB = 16
CHUNK = 128
N_PAD = 10240
ROWS_PER_SUB = N_PAD // NSUB
JUNK = N + 100
E_PAD = -(-E // (NSUB * CHUNK)) * (NSUB * CHUNK)
NCH = E_PAD // CHUNK
CPS = NCH // NSUB
PCHUNKS = N_PAD // CHUNK


def _sc_mesh():
    return plsc.VectorSubcoreMesh(
        core_axis_name="c", subcore_axis_name="s", num_cores=G
    )


def _spmm(table, src_r, dst_r):
    @functools.partial(
        pl.kernel,
        out_type=jax.ShapeDtypeStruct((G, N_PAD, D), jnp.float32),
        mesh=_sc_mesh(),
        scratch_types=[
            pltpu.VMEM((CHUNK,), jnp.int32),
            pltpu.VMEM((CHUNK,), jnp.int32),
            pltpu.VMEM((CHUNK, D), jnp.float32),
            pltpu.VMEM_SHARED((N_PAD, D), jnp.float32),
            pltpu.SemaphoreType.DMA,
        ],
    )
    def k(table_ref, src_ref, dst_ref, out_ref, idx_s, idx_d, r0,
          acc, g0):
        c = lax.axis_index("c")
        s = lax.axis_index("s")

        def zrow(r, carry):
            for j in range(D // 16):
                r0[r, pl.ds(j * 16, 16)] = jnp.zeros((16,), jnp.float32)
            return carry

        lax.fori_loop(0, CHUNK, zrow, 0)
        for t in range(ROWS_PER_SUB // CHUNK):
            pltpu.sync_copy(
                r0, acc.at[pl.ds(s * ROWS_PER_SUB + t * CHUNK, CHUNK)]
            )
        plsc.subcore_barrier()

        for t in range(CPS):
            ch = s * CPS + t
            pltpu.sync_copy(src_ref.at[c, ch], idx_s)
            pltpu.sync_copy(dst_ref.at[c, ch], idx_d)
            pltpu.async_copy(table_ref.at[idx_s], r0, g0)
            pltpu.make_async_copy(
                table_ref.at[pl.ds(0, CHUNK)], r0, g0
            ).wait()
            pltpu.sync_copy(r0, acc.at[idx_d], add=True)
        plsc.subcore_barrier()
        pltpu.sync_copy(
            acc.at[pl.ds(s * ROWS_PER_SUB, ROWS_PER_SUB)],
            out_ref.at[c, pl.ds(s * ROWS_PER_SUB, ROWS_PER_SUB)],
        )

    return k(table, src_r, dst_r)


def _perm_gather(xt, ht, perm_r):
    NFULL = N // CHUNK
    NTAIL = N - NFULL * CHUNK

    @functools.partial(
        pl.kernel,
        out_type=(
            jax.ShapeDtypeStruct((N_PAD, D), jnp.float32),
            jax.ShapeDtypeStruct((N_PAD, D), jnp.float32),
            jax.ShapeDtypeStruct((N_PAD, D), jnp.float32),
        ),
        mesh=_sc_mesh(),
        scratch_types=[
            pltpu.VMEM((CHUNK,), jnp.int32),
            pltpu.VMEM((CHUNK, D), jnp.float32),
            pltpu.SemaphoreType.DMA,
        ],
    )
    def k(x_ref, h_ref, perm_ref, xp_ref, hp_ref, xlin_ref, idx, rows, sem):
        c = lax.axis_index("c")
        s = lax.axis_index("s")
        w = s * G + c
        for t in range(-(-PCHUNKS // (2 * NSUB))):
            cid = w + 2 * NSUB * t

            @pl.when(cid < PCHUNKS)
            def _():
                pltpu.sync_copy(perm_ref.at[cid], idx)
                pltpu.async_copy(x_ref.at[idx], rows, sem).wait()
                pltpu.sync_copy(rows, xp_ref.at[pl.ds(cid * CHUNK, CHUNK)])
                pltpu.async_copy(h_ref.at[idx], rows, sem).wait()
                pltpu.sync_copy(rows, hp_ref.at[pl.ds(cid * CHUNK, CHUNK)])

            @pl.when(cid < NFULL)
            def _():
                pltpu.sync_copy(x_ref.at[pl.ds(cid * CHUNK, CHUNK)], rows)
                pltpu.sync_copy(rows, xlin_ref.at[pl.ds(cid * CHUNK, CHUNK)])

            @pl.when(cid == NFULL)
            def _():
                pltpu.sync_copy(
                    x_ref.at[pl.ds(NFULL * CHUNK, NTAIL)],
                    rows.at[pl.ds(0, NTAIL)],
                )
                pltpu.sync_copy(
                    rows.at[pl.ds(0, NTAIL)],
                    xlin_ref.at[pl.ds(NFULL * CHUNK, NTAIL)],
                )

    return k(xt, ht, perm_r)


def _fc(feature, W1):
    blk = 2000

    def body(f_ref, w_ref, o_ref):
        o_ref[...] = jnp.dot(
            f_ref[...], w_ref[...], preferred_element_type=jnp.float32
        )

    return pl.pallas_call(
        body,
        grid=(N // blk,),
        in_specs=[
            pl.BlockSpec((blk, D), lambda i: (i, 0)),
            pl.BlockSpec((D, D), lambda i: (0, 0)),
        ],
        out_specs=pl.BlockSpec((blk, D), lambda i: (i, 0)),
        out_shape=jax.ShapeDtypeStruct((N, D), jnp.float32),
    )(feature, W1)


def _losses(temp, x, h1, h2, zp, xp, H, hp):
    blk = 2000
    grid_n = N // blk

    def body(t_ref, x_ref, h1a, h1b, h2a, h2b, zpa, zpb, xp_ref, hh_ref,
             hp_ref, loss_ref, reg_ref):
        i = pl.program_id(0)

        @pl.when(i == 0)
        def _():
            loss_ref[0, 0] = 0.0
            reg_ref[0, 0] = 0.0

        xv = x_ref[...]
        xpv = xp_ref[...]
        hh = hh_ref[...]
        hpv = hp_ref[...]
        dHf = jnp.sum((hh - hpv) ** 2, axis=1)
        lsum = jnp.float32(0.0)
        berns = []
        for g, (h1r, h2r, zpr) in enumerate(
            ((h1a, h2a, zpa), (h1b, h2b, zpb))
        ):
            a0 = jnp.maximum(t_ref[g, 0], 0.0) * 0.25
            a1 = jnp.maximum(t_ref[g, 1], 0.0) * 0.5
            a2 = jnp.maximum(t_ref[g, 2], 0.0) * 0.25
            bern = a0 * xv + a1 * h1r[0] + a2 * h2r[0]
            d1p = jnp.sum((bern - zpr[0]) ** 2, axis=1)
            d1n = jnp.sum((bern - xpv) ** 2, axis=1)
            lsum += jnp.sum(jnp.maximum(d1p - d1n + ALPHA, 0.0))
            d2p = jnp.sum((hh - bern) ** 2, axis=1)
            lsum += jnp.sum(jnp.maximum(d2p - dHf + BETA, 0.0))
            berns.append(bern)
        pos = 0.5 * (berns[0] + berns[1])
        loss_ref[0, 0] += lsum * jnp.float32(1.0 / N)
        reg_ref[0, 0] += jnp.sum((hh - pos) ** 2)

    g3 = lambda i: (0, i, 0)
    g3b = lambda i: (1, i, 0)
    loss2, reg2 = pl.pallas_call(
        body,
        grid=(grid_n,),
        in_specs=[
            pl.BlockSpec(memory_space=pltpu.SMEM),
            pl.BlockSpec((blk, D), lambda i: (i, 0)),
            pl.BlockSpec((1, blk, D), g3),
            pl.BlockSpec((1, blk, D), g3b),
            pl.BlockSpec((1, blk, D), g3),
            pl.BlockSpec((1, blk, D), g3b),
            pl.BlockSpec((1, blk, D), g3),
            pl.BlockSpec((1, blk, D), g3b),
            pl.BlockSpec((blk, D), lambda i: (i, 0)),
            pl.BlockSpec((blk, D), lambda i: (i, 0)),
            pl.BlockSpec((blk, D), lambda i: (i, 0)),
        ],
        out_specs=[
            pl.BlockSpec((1, 1), lambda i: (0, 0), memory_space=pltpu.SMEM),
            pl.BlockSpec((1, 1), lambda i: (0, 0), memory_space=pltpu.SMEM),
        ],
        out_shape=[
            jax.ShapeDtypeStruct((1, 1), jnp.float32),
            jax.ShapeDtypeStruct((1, 1), jnp.float32),
        ],
        compiler_params=pltpu.CompilerParams(
            dimension_semantics=("arbitrary",)
        ),
    )(temp, x, h1, h1, h2, h2, zp, zp, xp, H, hp)
    return loss2[0, 0], reg2[0, 0]


def _prep_edges(ei, src_off):
    src = ei[:, 0, :].astype(jnp.int32)
    dst = ei[:, 1, :].astype(jnp.int32)
    pad = E_PAD - E
    src = jnp.concatenate([src, jnp.zeros((G, pad), jnp.int32)], axis=1)
    dst = jnp.concatenate([dst, jnp.full((G, pad), JUNK, jnp.int32)], axis=1)
    if src_off:
        src = src + (jnp.arange(G, dtype=jnp.int32) * N_PAD)[:, None]
    return src.reshape(G, NCH, CHUNK), dst.reshape(G, NCH, CHUNK)


def kernel(feature, adj, neighbor_adj, sparse, msk, samp_bias1, samp_bias2,
           W1, temp, H, perm_idx):
    src_r, dst_r = _prep_edges(adj, False)
    src2_r, _ = _prep_edges(adj, True)
    nsrc_r, ndst_r = _prep_edges(neighbor_adj, False)
    perm_r = jnp.concatenate(
        [perm_idx.astype(jnp.int32), jnp.zeros((N_PAD - N,), jnp.int32)]
    ).reshape(PCHUNKS, CHUNK)

    x = _fc(feature, W1)
    xp, hp, xlin = _perm_gather(x, H, perm_r)
    h1 = _spmm(xlin, src_r, dst_r)
    zp = _spmm(xlin, nsrc_r, ndst_r)
    h2 = _spmm(h1.reshape(G * N_PAD, D), src2_r, dst_r)
    loss, reg = _losses(temp, x, h1, h2, zp, xp, H, hp)
    return loss, reg


# speedup vs baseline: 2.0550x; 1.4271x over previous
import functools

import jax
import jax.numpy as jnp
from jax import lax
from jax.experimental import pallas as pl
from jax.experimental.pallas import tpu as pltpu
from jax.experimental.pallas import tpu_sc as plsc

N = 10000
D = 128
E = 320000
G = 2
ALPHA = 0.5
BETA = 0.5

NSUB = 16
CHUNK = 128
N_PAD = 10240
ROWS_PER_SUB = N_PAD // NSUB
JUNK = N + 100
E_PAD = -(-E // (NSUB * CHUNK)) * (NSUB * CHUNK)
NCH = E_PAD // CHUNK
CPS = NCH // NSUB
PCHUNKS = N_PAD // CHUNK


def _sc_mesh():
    return plsc.VectorSubcoreMesh(
        core_axis_name="c", subcore_axis_name="s", num_cores=G
    )


def _spmm(table, src_r, dst_r):
    @functools.partial(
        pl.kernel,
        out_type=jax.ShapeDtypeStruct((G, N_PAD, D), jnp.float32),
        mesh=_sc_mesh(),
        scratch_types=[
            pltpu.VMEM((CHUNK,), jnp.int32),
            pltpu.VMEM((CHUNK,), jnp.int32),
            pltpu.VMEM((CHUNK,), jnp.int32),
            pltpu.VMEM((CHUNK,), jnp.int32),
            pltpu.VMEM((CHUNK, D), jnp.float32),
            pltpu.VMEM((CHUNK, D), jnp.float32),
            pltpu.VMEM_SHARED((N_PAD, D), jnp.float32),
            pltpu.SemaphoreType.DMA,
            pltpu.SemaphoreType.DMA,
        ],
    )
    def k(table_ref, src_ref, dst_ref, out_ref, idx_s0, idx_d0,
          idx_s1, idx_d1, r0, r1, acc, g0, g1):
        c = lax.axis_index("c")
        s = lax.axis_index("s")

        def zrow(r, carry):
            for j in range(D // 16):
                r0[r, pl.ds(j * 16, 16)] = jnp.zeros((16,), jnp.float32)
            return carry

        lax.fori_loop(0, CHUNK, zrow, 0)
        for t in range(ROWS_PER_SUB // CHUNK):
            pltpu.sync_copy(
                r0, acc.at[pl.ds(s * ROWS_PER_SUB + t * CHUNK, CHUNK)]
            )
        plsc.subcore_barrier()

        bufs = ((idx_s0, idx_d0, r0, g0), (idx_s1, idx_d1, r1, g1))

        def load_and_gather(t, b):
            ch = s * CPS + t
            pltpu.sync_copy(src_ref.at[c, ch], b[0])
            pltpu.sync_copy(dst_ref.at[c, ch], b[1])
            pltpu.async_copy(table_ref.at[b[0]], b[2], b[3])

        load_and_gather(0, bufs[0])
        for t in range(CPS):
            cur = bufs[t % 2]
            if t + 1 < CPS:
                load_and_gather(t + 1, bufs[(t + 1) % 2])
            pltpu.make_async_copy(
                table_ref.at[pl.ds(0, CHUNK)], cur[2], cur[3]
            ).wait()
            pltpu.sync_copy(cur[2], acc.at[cur[1]], add=True)
        plsc.subcore_barrier()
        pltpu.sync_copy(
            acc.at[pl.ds(s * ROWS_PER_SUB, ROWS_PER_SUB)],
            out_ref.at[c, pl.ds(s * ROWS_PER_SUB, ROWS_PER_SUB)],
        )

    return k(table, src_r, dst_r)


def _perm_gather(xt, ht, perm_r):
    NFULL = N // CHUNK
    NTAIL = N - NFULL * CHUNK

    @functools.partial(
        pl.kernel,
        out_type=(
            jax.ShapeDtypeStruct((N_PAD, D), jnp.float32),
            jax.ShapeDtypeStruct((N_PAD, D), jnp.float32),
            jax.ShapeDtypeStruct((N_PAD, D), jnp.float32),
        ),
        mesh=_sc_mesh(),
        scratch_types=[
            pltpu.VMEM((CHUNK,), jnp.int32),
            pltpu.VMEM((CHUNK, D), jnp.float32),
            pltpu.SemaphoreType.DMA,
        ],
    )
    def k(x_ref, h_ref, perm_ref, xp_ref, hp_ref, xlin_ref, idx, rows, sem):
        c = lax.axis_index("c")
        s = lax.axis_index("s")
        w = s * G + c
        for t in range(-(-PCHUNKS // (2 * NSUB))):
            cid = w + 2 * NSUB * t

            @pl.when(cid < PCHUNKS)
            def _():
                pltpu.sync_copy(perm_ref.at[cid], idx)
                pltpu.async_copy(x_ref.at[idx], rows, sem).wait()
                pltpu.sync_copy(rows, xp_ref.at[pl.ds(cid * CHUNK, CHUNK)])
                pltpu.async_copy(h_ref.at[idx], rows, sem).wait()
                pltpu.sync_copy(rows, hp_ref.at[pl.ds(cid * CHUNK, CHUNK)])

            @pl.when(cid < NFULL)
            def _():
                pltpu.sync_copy(x_ref.at[pl.ds(cid * CHUNK, CHUNK)], rows)
                pltpu.sync_copy(rows, xlin_ref.at[pl.ds(cid * CHUNK, CHUNK)])

            @pl.when(cid == NFULL)
            def _():
                pltpu.sync_copy(
                    x_ref.at[pl.ds(NFULL * CHUNK, NTAIL)],
                    rows.at[pl.ds(0, NTAIL)],
                )
                pltpu.sync_copy(
                    rows.at[pl.ds(0, NTAIL)],
                    xlin_ref.at[pl.ds(NFULL * CHUNK, NTAIL)],
                )

    return k(xt, ht, perm_r)


def _fc(feature, W1):
    blk = 2000

    def body(f_ref, w_ref, o_ref):
        o_ref[...] = jnp.dot(
            f_ref[...], w_ref[...], preferred_element_type=jnp.float32
        )

    return pl.pallas_call(
        body,
        grid=(N // blk,),
        in_specs=[
            pl.BlockSpec((blk, D), lambda i: (i, 0)),
            pl.BlockSpec((D, D), lambda i: (0, 0)),
        ],
        out_specs=pl.BlockSpec((blk, D), lambda i: (i, 0)),
        out_shape=jax.ShapeDtypeStruct((N, D), jnp.float32),
    )(feature, W1)


def _losses(temp, x, h1, h2, zp, xp, H, hp):
    blk = 2000
    grid_n = N // blk

    def body(t_ref, x_ref, h1a, h1b, h2a, h2b, zpa, zpb, xp_ref, hh_ref,
             hp_ref, loss_ref, reg_ref):
        i = pl.program_id(0)

        @pl.when(i == 0)
        def _():
            loss_ref[0, 0] = 0.0
            reg_ref[0, 0] = 0.0

        xv = x_ref[...]
        xpv = xp_ref[...]
        hh = hh_ref[...]
        hpv = hp_ref[...]
        dHf = jnp.sum((hh - hpv) ** 2, axis=1)
        lsum = jnp.float32(0.0)
        berns = []
        for g, (h1r, h2r, zpr) in enumerate(
            ((h1a, h2a, zpa), (h1b, h2b, zpb))
        ):
            a0 = jnp.maximum(t_ref[g, 0], 0.0) * 0.25
            a1 = jnp.maximum(t_ref[g, 1], 0.0) * 0.5
            a2 = jnp.maximum(t_ref[g, 2], 0.0) * 0.25
            bern = a0 * xv + a1 * h1r[0] + a2 * h2r[0]
            d1p = jnp.sum((bern - zpr[0]) ** 2, axis=1)
            d1n = jnp.sum((bern - xpv) ** 2, axis=1)
            lsum += jnp.sum(jnp.maximum(d1p - d1n + ALPHA, 0.0))
            d2p = jnp.sum((hh - bern) ** 2, axis=1)
            lsum += jnp.sum(jnp.maximum(d2p - dHf + BETA, 0.0))
            berns.append(bern)
        pos = 0.5 * (berns[0] + berns[1])
        loss_ref[0, 0] += lsum * jnp.float32(1.0 / N)
        reg_ref[0, 0] += jnp.sum((hh - pos) ** 2)

    g3 = lambda i: (0, i, 0)
    g3b = lambda i: (1, i, 0)
    loss2, reg2 = pl.pallas_call(
        body,
        grid=(grid_n,),
        in_specs=[
            pl.BlockSpec(memory_space=pltpu.SMEM),
            pl.BlockSpec((blk, D), lambda i: (i, 0)),
            pl.BlockSpec((1, blk, D), g3),
            pl.BlockSpec((1, blk, D), g3b),
            pl.BlockSpec((1, blk, D), g3),
            pl.BlockSpec((1, blk, D), g3b),
            pl.BlockSpec((1, blk, D), g3),
            pl.BlockSpec((1, blk, D), g3b),
            pl.BlockSpec((blk, D), lambda i: (i, 0)),
            pl.BlockSpec((blk, D), lambda i: (i, 0)),
            pl.BlockSpec((blk, D), lambda i: (i, 0)),
        ],
        out_specs=[
            pl.BlockSpec((1, 1), lambda i: (0, 0), memory_space=pltpu.SMEM),
            pl.BlockSpec((1, 1), lambda i: (0, 0), memory_space=pltpu.SMEM),
        ],
        out_shape=[
            jax.ShapeDtypeStruct((1, 1), jnp.float32),
            jax.ShapeDtypeStruct((1, 1), jnp.float32),
        ],
        compiler_params=pltpu.CompilerParams(
            dimension_semantics=("arbitrary",)
        ),
    )(temp, x, h1, h1, h2, h2, zp, zp, xp, H, hp)
    return loss2[0, 0], reg2[0, 0]


def _prep_edges(ei, src_off):
    src = ei[:, 0, :].astype(jnp.int32)
    dst = ei[:, 1, :].astype(jnp.int32)
    pad = E_PAD - E
    src = jnp.concatenate([src, jnp.zeros((G, pad), jnp.int32)], axis=1)
    dst = jnp.concatenate([dst, jnp.full((G, pad), JUNK, jnp.int32)], axis=1)
    if src_off:
        src = src + (jnp.arange(G, dtype=jnp.int32) * N_PAD)[:, None]
    return src.reshape(G, NCH, CHUNK), dst.reshape(G, NCH, CHUNK)


def kernel(feature, adj, neighbor_adj, sparse, msk, samp_bias1, samp_bias2,
           W1, temp, H, perm_idx):
    src_r, dst_r = _prep_edges(adj, False)
    src2_r, _ = _prep_edges(adj, True)
    nsrc_r, ndst_r = _prep_edges(neighbor_adj, False)
    perm_r = jnp.concatenate(
        [perm_idx.astype(jnp.int32), jnp.zeros((N_PAD - N,), jnp.int32)]
    ).reshape(PCHUNKS, CHUNK)

    x = _fc(feature, W1)
    xp, hp, xlin = _perm_gather(x, H, perm_r)
    h1 = _spmm(xlin, src_r, dst_r)
    zp = _spmm(xlin, nsrc_r, ndst_r)
    h2 = _spmm(h1.reshape(G * N_PAD, D), src2_r, dst_r)
    loss, reg = _losses(temp, x, h1, h2, zp, xp, H, hp)
    return loss, reg
